# Initial kernel scaffold; baseline (speedup 1.0000x reference)
import jax
import jax.numpy as jnp
from jax.experimental import pallas as pl


def kernel(user_idx, item_idx, fav_subjects, book_subjects, subj_emb, attn_w, attn_b, user_bias, item_bias, global_bias):
    raise NotImplementedError("write your pallas kernel here")


# speedup vs baseline: 10.8869x; 10.8869x over previous
import functools

import jax
import jax.numpy as jnp
from jax import lax
from jax.experimental import pallas as pl
from jax.experimental.pallas import tpu as pltpu
from jax.experimental.pallas import tpu_sc as plsc

_NC = 2
_NS = 16
_NW = _NC * _NS
_LANES = 16
_L = 50
_D = 16
_CH = 64
_NIDX = _CH * _L
_IDXW = 128
_G = _NIDX // _IDXW


def _splat(x):
  return jnp.broadcast_to(x, (_LANES,))


def _sc_body(user_idx, item_idx, fav_r, book_r, subj_emb, attn_w, consts,
             z16, user_bias, item_bias, out,
             idxf, idxb, rows_f, rows_b, sc, pool_f, uidx, iidx, uidxs, iidxs,
             ub, ib, outc, attnw_v, consts_v, z_v, sem_f, sem_b, sem_s):
  wid = lax.axis_index("s") * _NC + lax.axis_index("c")
  rows_pw = out.shape[0] // _NW
  n_chunks = rows_pw // _CH
  iota = jnp.arange(_LANES, dtype=jnp.int32)

  pltpu.sync_copy(z16, z_v)
  zv = z_v[...]
  pltpu.sync_copy(attn_w, attnw_v.at[pl.ds(_LANES, _LANES)])
  pltpu.sync_copy(consts, consts_v.at[pl.ds(8, 2)])
  awk = [plsc.load_gather(attnw_v, [jnp.full((_LANES,), _LANES + k, jnp.int32)])
         for k in range(_D)]
  b_sp = plsc.load_gather(consts_v, [jnp.full((_LANES,), 8, jnp.int32)])
  gb_sp = plsc.load_gather(consts_v, [jnp.full((_LANES,), 9, jnp.int32)])
  neg_inf = jnp.float32(-jnp.inf)

  def scores_for(rows_l):
    def grp(t, c):
      s = b_sp
      for k in range(_D):
        kidx = zv if k == 0 else jnp.full((_LANES,), k, jnp.int32)
        col = plsc.load_gather(rows_l, [t * _LANES + iota, kidx])
        s = s + col * awk[k]
      sc[pl.ds(t * _LANES, _LANES)] = s
      return c
    lax.fori_loop(0, _NIDX // _LANES, grp, 0)

  def pool_row(r, idx_l, rows_l):
    base = r * _L
    qs, svs, valids = [], [], []
    for g in range(4):
      qv = base + g * _LANES + iota
      if g == 3:
        lane_ok = iota < (_L - 3 * _LANES)
        qv = jnp.where(lane_ok, qv, base)
      idv = plsc.load_gather(idx_l, [qv])
      sv = plsc.load_gather(sc, [qv])
      valid = idv != 0
      if g == 3:
        valid = valid & lane_ok
      qs.append(qv)
      svs.append(sv)
      valids.append(valid)
    pc = (plsc.all_reduce_population_count(valids[0]) +
          plsc.all_reduce_population_count(valids[1]) +
          plsc.all_reduce_population_count(valids[2]) +
          plsc.all_reduce_population_count(valids[3]))
    has_real = pc > 0
    valids[0] = jnp.where(has_real, valids[0], iota == 0)
    ms = [jnp.where(valids[g], svs[g], neg_inf) for g in range(4)]
    m = jnp.max(jnp.maximum(jnp.maximum(ms[0], ms[1]),
                            jnp.maximum(ms[2], ms[3])))
    es = [jnp.exp(ms[g] - m) for g in range(4)]
    den = jnp.sum(es[0] + es[1] + es[2] + es[3])
    denv = _splat(den)
    for g in range(4):
      w = es[g] / denv
      if g == 3:
        plsc.store_scatter(sc, [qs[g]], w, mask=iota < (_L - 3 * _LANES))
      else:
        plsc.store_scatter(sc, [qs[g]], w)

    def pacc(l, p):
      q = base + l
      qsp = _splat(q)
      wsp = plsc.load_gather(sc, [qsp])
      emb = plsc.load_gather(rows_l, [qsp, iota])
      return p + wsp * emb
    return lax.fori_loop(0, _L, pacc, jnp.zeros((_LANES,), jnp.float32))

  def chunk(step, carry):
    rbase = wid * rows_pw + step * _CH
    ibase = rbase * _L
    pltpu.sync_copy(fav_r.at[pl.ds(ibase, _NIDX)], idxf)
    pltpu.sync_copy(book_r.at[pl.ds(ibase, _NIDX)], idxb)
    pltpu.sync_copy(user_idx.at[pl.ds(rbase, _CH)], uidx)
    pltpu.sync_copy(item_idx.at[pl.ds(rbase, _CH)], iidx)
    for i in range(_CH // _LANES):
      uidxs[pl.ds(i * _LANES, _LANES)] = lax.shift_right_logical(
          uidx[pl.ds(i * _LANES, _LANES)], 4)
      iidxs[pl.ds(i * _LANES, _LANES)] = lax.shift_right_logical(
          iidx[pl.ds(i * _LANES, _LANES)], 4)
    def fire_f(g, c):
      pltpu.make_async_copy(subj_emb.at[idxf.at[pl.ds(g * _IDXW, _IDXW)]],
                            rows_f.at[pl.ds(g * _IDXW, _IDXW)], sem_f).start()
      return c
    lax.fori_loop(0, _G, fire_f, 0)
    def fire_b(g, c):
      pltpu.make_async_copy(subj_emb.at[idxb.at[pl.ds(g * _IDXW, _IDXW)]],
                            rows_b.at[pl.ds(g * _IDXW, _IDXW)], sem_b).start()
      return c
    lax.fori_loop(0, _G, fire_b, 0)
    pltpu.make_async_copy(user_bias.at[uidxs], ub, sem_s).start()
    pltpu.make_async_copy(item_bias.at[iidxs], ib, sem_s).start()

    def drain_f(g, c):
      pltpu.make_async_copy(subj_emb.at[idxf.at[pl.ds(g * _IDXW, _IDXW)]],
                            rows_f.at[pl.ds(g * _IDXW, _IDXW)], sem_f).wait()
      return c
    lax.fori_loop(0, _G, drain_f, 0)
    scores_for(rows_f)
    def rowf(r, c):
      p = pool_row(r, idxf, rows_f)
      plsc.store_scatter(pool_f, [_splat(r), iota], p)
      return c
    lax.fori_loop(0, _CH, rowf, 0)

    def drain_b(g, c):
      pltpu.make_async_copy(subj_emb.at[idxb.at[pl.ds(g * _IDXW, _IDXW)]],
                            rows_b.at[pl.ds(g * _IDXW, _IDXW)], sem_b).wait()
      return c
    lax.fori_loop(0, _G, drain_b, 0)
    scores_for(rows_b)
    def rowb(r, c):
      pb = pool_row(r, idxb, rows_b)
      pf = plsc.load_gather(pool_f, [_splat(r), iota])
      d = jnp.sum(pf * pb)
      plsc.store_scatter(outc, [_splat(r)], _splat(d), mask=iota == 0)
      return c
    lax.fori_loop(0, _CH, rowb, 0)

    pltpu.make_async_copy(user_bias.at[uidxs], ub, sem_s).wait()
    pltpu.make_async_copy(item_bias.at[iidxs], ib, sem_s).wait()
    for i in range(_CH // _LANES):
      ov = outc[pl.ds(i * _LANES, _LANES)]
      ulane = lax.bitwise_and(uidx[pl.ds(i * _LANES, _LANES)], 15)
      ilane = lax.bitwise_and(iidx[pl.ds(i * _LANES, _LANES)], 15)
      ubv = plsc.load_gather(ub, [i * _LANES + iota, ulane])
      ibv = plsc.load_gather(ib, [i * _LANES + iota, ilane])
      outc[pl.ds(i * _LANES, _LANES)] = ov + ubv + ibv + gb_sp
    pltpu.sync_copy(outc, out.at[pl.ds(rbase, _CH)])
    return carry

  lax.fori_loop(0, n_chunks, chunk, 0)


def kernel(user_idx, item_idx, fav_subjects, book_subjects, subj_emb, attn_w,
           attn_b, user_bias, item_bias, global_bias):
  B = user_idx.shape[0]
  fav_r = fav_subjects.astype(jnp.int32).reshape(-1)
  book_r = book_subjects.astype(jnp.int32).reshape(-1)
  consts = jnp.concatenate([
      jnp.reshape(attn_b, (1,)).astype(jnp.float32),
      jnp.reshape(global_bias, (1,)).astype(jnp.float32),
  ])
  mesh = plsc.VectorSubcoreMesh(core_axis_name="c", subcore_axis_name="s",
                                num_cores=_NC, num_subcores=_NS)
  run = functools.partial(
      pl.kernel,
      out_type=jax.ShapeDtypeStruct((B,), jnp.float32),
      mesh=mesh,
      compiler_params=pltpu.CompilerParams(needs_layout_passes=False,
                                           use_tc_tiling_on_sc=False),
      scratch_types=[
          pltpu.VMEM((_NIDX,), jnp.int32),
          pltpu.VMEM((_NIDX,), jnp.int32),
          pltpu.VMEM((_NIDX, _D), jnp.float32),
          pltpu.VMEM((_NIDX, _D), jnp.float32),
          pltpu.VMEM((_NIDX,), jnp.float32),
          pltpu.VMEM((_CH, _D), jnp.float32),
          pltpu.VMEM((_CH,), jnp.int32),
          pltpu.VMEM((_CH,), jnp.int32),
          pltpu.VMEM((_CH,), jnp.int32),
          pltpu.VMEM((_CH,), jnp.int32),
          pltpu.VMEM((_CH, _LANES), jnp.float32),
          pltpu.VMEM((_CH, _LANES), jnp.float32),
          pltpu.VMEM((_CH,), jnp.float32),
          pltpu.VMEM((2 * _LANES,), jnp.float32),
          pltpu.VMEM((_LANES,), jnp.float32),
          pltpu.VMEM((_LANES,), jnp.int32),
          pltpu.SemaphoreType.DMA,
          pltpu.SemaphoreType.DMA,
          pltpu.SemaphoreType.DMA,
      ],
  )(_sc_body)
  z16 = jnp.zeros((_LANES,), jnp.int32)
  ub_r = user_bias.astype(jnp.float32).reshape(-1, _LANES)
  ib_r = item_bias.astype(jnp.float32).reshape(-1, _LANES)
  return run(user_idx.astype(jnp.int32), item_idx.astype(jnp.int32),
             fav_r, book_r, subj_emb.astype(jnp.float32),
             attn_w.astype(jnp.float32), consts, z16, ub_r, ib_r)


# speedup vs baseline: 13.2822x; 1.2200x over previous
import functools

import jax
import jax.numpy as jnp
from jax import lax
from jax.experimental import pallas as pl
from jax.experimental.pallas import tpu as pltpu
from jax.experimental.pallas import tpu_sc as plsc

_NC = 2
_NS = 16
_NW = _NC * _NS
_LANES = 16
_L = 50
_D = 16
_CH = 64
_NIDX = _CH * _L
_IDXW = 128
_G = _NIDX // _IDXW


def _splat(x):
  return jnp.broadcast_to(x, (_LANES,))


def _sc_body(user_idx, item_idx, fav_r, book_r, subj_emb, attn_w, consts,
             z16, user_bias, item_bias, out,
             idxf, idxb, rows_f, rows_b, sc, pool_f, uidx, iidx, uidxs, iidxs,
             ub, ib, outc, attnw_v, consts_v, z_v, sem_f, sem_b, sem_s):
  wid = lax.axis_index("s") * _NC + lax.axis_index("c")
  rows_pw = out.shape[0] // _NW
  n_chunks = rows_pw // _CH
  iota = jnp.arange(_LANES, dtype=jnp.int32)

  pltpu.sync_copy(z16, z_v)
  zv = z_v[...]
  pltpu.sync_copy(attn_w, attnw_v.at[pl.ds(_LANES, _LANES)])
  pltpu.sync_copy(consts, consts_v.at[pl.ds(8, 2)])
  awk = [plsc.load_gather(attnw_v, [jnp.full((_LANES,), _LANES + k, jnp.int32)])
         for k in range(_D)]
  b_sp = plsc.load_gather(consts_v, [jnp.full((_LANES,), 8, jnp.int32)])
  gb_sp = plsc.load_gather(consts_v, [jnp.full((_LANES,), 9, jnp.int32)])
  neg_inf = jnp.float32(-jnp.inf)

  def scores_for(rows_l):
    def grp(t, c):
      s = b_sp
      for k in range(_D):
        kidx = zv if k == 0 else jnp.full((_LANES,), k, jnp.int32)
        col = plsc.load_gather(rows_l, [t * _LANES + iota, kidx])
        s = s + col * awk[k]
      sc[pl.ds(t * _LANES, _LANES)] = s
      return c
    lax.fori_loop(0, _NIDX // _LANES, grp, 0)

  def pool_row(r, idx_l, rows_l):
    base = r * _L
    qs, svs, valids = [], [], []
    for g in range(4):
      qv = base + g * _LANES + iota
      if g == 3:
        lane_ok = iota < (_L - 3 * _LANES)
        qv = jnp.where(lane_ok, qv, base)
      idv = plsc.load_gather(idx_l, [qv])
      sv = plsc.load_gather(sc, [qv])
      valid = idv != 0
      if g == 3:
        valid = valid & lane_ok
      qs.append(qv)
      svs.append(sv)
      valids.append(valid)
    pc = (plsc.all_reduce_population_count(valids[0]) +
          plsc.all_reduce_population_count(valids[1]) +
          plsc.all_reduce_population_count(valids[2]) +
          plsc.all_reduce_population_count(valids[3]))
    has_real = pc > 0
    valids[0] = jnp.where(has_real, valids[0], iota == 0)
    ms = [jnp.where(valids[g], svs[g], neg_inf) for g in range(4)]
    m = jnp.max(jnp.maximum(jnp.maximum(ms[0], ms[1]),
                            jnp.maximum(ms[2], ms[3])))
    es = [jnp.exp(ms[g] - m) for g in range(4)]
    den = jnp.sum(es[0] + es[1] + es[2] + es[3])
    denv = _splat(den)
    for g in range(4):
      w = es[g] / denv
      if g == 3:
        plsc.store_scatter(sc, [qs[g]], w, mask=iota < (_L - 3 * _LANES))
      else:
        plsc.store_scatter(sc, [qs[g]], w)

    def pacc(l5, p):
      for j in range(5):
        q = base + l5 * 5 + j
        qsp = _splat(q)
        wsp = plsc.load_gather(sc, [qsp])
        emb = plsc.load_gather(rows_l, [qsp, iota])
        p = p + wsp * emb
      return p
    return lax.fori_loop(0, _L // 5, pacc, jnp.zeros((_LANES,), jnp.float32))

  def chunk(step, carry):
    rbase = wid * rows_pw + step * _CH
    ibase = rbase * _L
    pltpu.sync_copy(fav_r.at[pl.ds(ibase, _NIDX)], idxf)
    pltpu.sync_copy(book_r.at[pl.ds(ibase, _NIDX)], idxb)
    pltpu.sync_copy(user_idx.at[pl.ds(rbase, _CH)], uidx)
    pltpu.sync_copy(item_idx.at[pl.ds(rbase, _CH)], iidx)
    for i in range(_CH // _LANES):
      uidxs[pl.ds(i * _LANES, _LANES)] = lax.shift_right_logical(
          uidx[pl.ds(i * _LANES, _LANES)], 4)
      iidxs[pl.ds(i * _LANES, _LANES)] = lax.shift_right_logical(
          iidx[pl.ds(i * _LANES, _LANES)], 4)
    def fire_f(g, c):
      pltpu.make_async_copy(subj_emb.at[idxf.at[pl.ds(g * _IDXW, _IDXW)]],
                            rows_f.at[pl.ds(g * _IDXW, _IDXW)], sem_f).start()
      return c
    lax.fori_loop(0, _G, fire_f, 0)
    def fire_b(g, c):
      pltpu.make_async_copy(subj_emb.at[idxb.at[pl.ds(g * _IDXW, _IDXW)]],
                            rows_b.at[pl.ds(g * _IDXW, _IDXW)], sem_b).start()
      return c
    lax.fori_loop(0, _G, fire_b, 0)
    pltpu.make_async_copy(user_bias.at[uidxs], ub, sem_s).start()
    pltpu.make_async_copy(item_bias.at[iidxs], ib, sem_s).start()

    def drain_f(g, c):
      pltpu.make_async_copy(subj_emb.at[idxf.at[pl.ds(g * _IDXW, _IDXW)]],
                            rows_f.at[pl.ds(g * _IDXW, _IDXW)], sem_f).wait()
      return c
    lax.fori_loop(0, _G, drain_f, 0)
    scores_for(rows_f)
    def rowf(r, c):
      p = pool_row(r, idxf, rows_f)
      plsc.store_scatter(pool_f, [_splat(r), iota], p)
      return c
    lax.fori_loop(0, _CH, rowf, 0)

    def drain_b(g, c):
      pltpu.make_async_copy(subj_emb.at[idxb.at[pl.ds(g * _IDXW, _IDXW)]],
                            rows_b.at[pl.ds(g * _IDXW, _IDXW)], sem_b).wait()
      return c
    lax.fori_loop(0, _G, drain_b, 0)
    scores_for(rows_b)
    def rowb(r, c):
      pb = pool_row(r, idxb, rows_b)
      pf = plsc.load_gather(pool_f, [_splat(r), iota])
      d = jnp.sum(pf * pb)
      plsc.store_scatter(outc, [_splat(r)], _splat(d), mask=iota == 0)
      return c
    lax.fori_loop(0, _CH, rowb, 0)

    pltpu.make_async_copy(user_bias.at[uidxs], ub, sem_s).wait()
    pltpu.make_async_copy(item_bias.at[iidxs], ib, sem_s).wait()
    for i in range(_CH // _LANES):
      ov = outc[pl.ds(i * _LANES, _LANES)]
      ulane = lax.bitwise_and(uidx[pl.ds(i * _LANES, _LANES)], 15)
      ilane = lax.bitwise_and(iidx[pl.ds(i * _LANES, _LANES)], 15)
      ubv = plsc.load_gather(ub, [i * _LANES + iota, ulane])
      ibv = plsc.load_gather(ib, [i * _LANES + iota, ilane])
      outc[pl.ds(i * _LANES, _LANES)] = ov + ubv + ibv + gb_sp
    pltpu.sync_copy(outc, out.at[pl.ds(rbase, _CH)])
    return carry

  lax.fori_loop(0, n_chunks, chunk, 0)


def kernel(user_idx, item_idx, fav_subjects, book_subjects, subj_emb, attn_w,
           attn_b, user_bias, item_bias, global_bias):
  B = user_idx.shape[0]
  fav_r = fav_subjects.astype(jnp.int32).reshape(-1)
  book_r = book_subjects.astype(jnp.int32).reshape(-1)
  consts = jnp.concatenate([
      jnp.reshape(attn_b, (1,)).astype(jnp.float32),
      jnp.reshape(global_bias, (1,)).astype(jnp.float32),
  ])
  mesh = plsc.VectorSubcoreMesh(core_axis_name="c", subcore_axis_name="s",
                                num_cores=_NC, num_subcores=_NS)
  run = functools.partial(
      pl.kernel,
      out_type=jax.ShapeDtypeStruct((B,), jnp.float32),
      mesh=mesh,
      compiler_params=pltpu.CompilerParams(needs_layout_passes=False,
                                           use_tc_tiling_on_sc=False),
      scratch_types=[
          pltpu.VMEM((_NIDX,), jnp.int32),
          pltpu.VMEM((_NIDX,), jnp.int32),
          pltpu.VMEM((_NIDX, _D), jnp.float32),
          pltpu.VMEM((_NIDX, _D), jnp.float32),
          pltpu.VMEM((_NIDX,), jnp.float32),
          pltpu.VMEM((_CH, _D), jnp.float32),
          pltpu.VMEM((_CH,), jnp.int32),
          pltpu.VMEM((_CH,), jnp.int32),
          pltpu.VMEM((_CH,), jnp.int32),
          pltpu.VMEM((_CH,), jnp.int32),
          pltpu.VMEM((_CH, _LANES), jnp.float32),
          pltpu.VMEM((_CH, _LANES), jnp.float32),
          pltpu.VMEM((_CH,), jnp.float32),
          pltpu.VMEM((2 * _LANES,), jnp.float32),
          pltpu.VMEM((_LANES,), jnp.float32),
          pltpu.VMEM((_LANES,), jnp.int32),
          pltpu.SemaphoreType.DMA,
          pltpu.SemaphoreType.DMA,
          pltpu.SemaphoreType.DMA,
      ],
  )(_sc_body)
  z16 = jnp.zeros((_LANES,), jnp.int32)
  ub_r = user_bias.astype(jnp.float32).reshape(-1, _LANES)
  ib_r = item_bias.astype(jnp.float32).reshape(-1, _LANES)
  return run(user_idx.astype(jnp.int32), item_idx.astype(jnp.int32),
             fav_r, book_r, subj_emb.astype(jnp.float32),
             attn_w.astype(jnp.float32), consts, z16, ub_r, ib_r)


# speedup vs baseline: 13.7204x; 1.0330x over previous
import functools

import jax
import jax.numpy as jnp
from jax import lax
from jax.experimental import pallas as pl
from jax.experimental.pallas import tpu as pltpu
from jax.experimental.pallas import tpu_sc as plsc

_NC = 2
_NS = 16
_NW = _NC * _NS
_LANES = 16
_L = 50
_D = 16
_CH = 64
_NIDX = _CH * _L
_IDXW = 128
_G = _NIDX // _IDXW


def _splat(x):
  return jnp.broadcast_to(x, (_LANES,))


def _sc_body(user_idx, item_idx, fav_r, book_r, subj_emb, attn_w, consts,
             z16, user_bias, item_bias, out,
             idxf, idxb, rows_f, rows_b, sc, pool_f, uidx, iidx, uidxs, iidxs,
             ub, ib, outc, attnw_v, consts_v, z_v, sem_f, sem_b, sem_s):
  wid = lax.axis_index("s") * _NC + lax.axis_index("c")
  rows_pw = out.shape[0] // _NW
  n_chunks = rows_pw // _CH
  iota = jnp.arange(_LANES, dtype=jnp.int32)

  pltpu.sync_copy(z16, z_v)
  zv = z_v[...]
  pltpu.sync_copy(attn_w, attnw_v.at[pl.ds(_LANES, _LANES)])
  pltpu.sync_copy(consts, consts_v.at[pl.ds(8, 2)])
  awk = [plsc.load_gather(attnw_v, [jnp.full((_LANES,), _LANES + k, jnp.int32)])
         for k in range(_D)]
  b_sp = plsc.load_gather(consts_v, [jnp.full((_LANES,), 8, jnp.int32)])
  gb_sp = plsc.load_gather(consts_v, [jnp.full((_LANES,), 9, jnp.int32)])
  neg_inf = jnp.float32(-jnp.inf)

  def scores_for(rows_l):
    def grp(t2, c):
      for u in range(2):
        t = t2 * 2 + u
        s = b_sp
        for k in range(_D):
          kidx = zv if k == 0 else jnp.full((_LANES,), k, jnp.int32)
          col = plsc.load_gather(rows_l, [t * _LANES + iota, kidx])
          s = s + col * awk[k]
        sc[pl.ds(t * _LANES, _LANES)] = s
      return c
    lax.fori_loop(0, _NIDX // (2 * _LANES), grp, 0)

  def pool_row(r, idx_l, rows_l):
    base = r * _L
    qs, svs, valids = [], [], []
    for g in range(4):
      qv = base + g * _LANES + iota
      if g == 3:
        lane_ok = iota < (_L - 3 * _LANES)
        qv = jnp.where(lane_ok, qv, base)
      idv = plsc.load_gather(idx_l, [qv])
      sv = plsc.load_gather(sc, [qv])
      valid = idv != 0
      if g == 3:
        valid = valid & lane_ok
      qs.append(qv)
      svs.append(sv)
      valids.append(valid)
    pc = (plsc.all_reduce_population_count(valids[0]) +
          plsc.all_reduce_population_count(valids[1]) +
          plsc.all_reduce_population_count(valids[2]) +
          plsc.all_reduce_population_count(valids[3]))
    has_real = pc > 0
    valids[0] = jnp.where(has_real, valids[0], iota == 0)
    ms = [jnp.where(valids[g], svs[g], neg_inf) for g in range(4)]
    m = jnp.max(jnp.maximum(jnp.maximum(ms[0], ms[1]),
                            jnp.maximum(ms[2], ms[3])))
    es = [jnp.exp(ms[g] - m) for g in range(4)]
    den = jnp.sum(es[0] + es[1] + es[2] + es[3])
    denv = _splat(den)
    for g in range(4):
      w = es[g] / denv
      if g == 3:
        plsc.store_scatter(sc, [qs[g]], w, mask=iota < (_L - 3 * _LANES))
      else:
        plsc.store_scatter(sc, [qs[g]], w)

    def pacc(l5, p):
      for j in range(10):
        q = base + l5 * 10 + j
        qsp = _splat(q)
        wsp = plsc.load_gather(sc, [qsp])
        emb = plsc.load_gather(rows_l, [qsp, iota])
        p = p + wsp * emb
      return p
    return lax.fori_loop(0, _L // 10, pacc, jnp.zeros((_LANES,), jnp.float32))

  def chunk(step, carry):
    rbase = wid * rows_pw + step * _CH
    ibase = rbase * _L
    pltpu.sync_copy(fav_r.at[pl.ds(ibase, _NIDX)], idxf)
    pltpu.sync_copy(book_r.at[pl.ds(ibase, _NIDX)], idxb)
    pltpu.sync_copy(user_idx.at[pl.ds(rbase, _CH)], uidx)
    pltpu.sync_copy(item_idx.at[pl.ds(rbase, _CH)], iidx)
    for i in range(_CH // _LANES):
      uidxs[pl.ds(i * _LANES, _LANES)] = lax.shift_right_logical(
          uidx[pl.ds(i * _LANES, _LANES)], 4)
      iidxs[pl.ds(i * _LANES, _LANES)] = lax.shift_right_logical(
          iidx[pl.ds(i * _LANES, _LANES)], 4)
    def fire_f(g, c):
      pltpu.make_async_copy(subj_emb.at[idxf.at[pl.ds(g * _IDXW, _IDXW)]],
                            rows_f.at[pl.ds(g * _IDXW, _IDXW)], sem_f).start()
      return c
    lax.fori_loop(0, _G, fire_f, 0)
    def fire_b(g, c):
      pltpu.make_async_copy(subj_emb.at[idxb.at[pl.ds(g * _IDXW, _IDXW)]],
                            rows_b.at[pl.ds(g * _IDXW, _IDXW)], sem_b).start()
      return c
    lax.fori_loop(0, _G, fire_b, 0)
    pltpu.make_async_copy(user_bias.at[uidxs], ub, sem_s).start()
    pltpu.make_async_copy(item_bias.at[iidxs], ib, sem_s).start()

    def drain_f(g, c):
      pltpu.make_async_copy(subj_emb.at[idxf.at[pl.ds(g * _IDXW, _IDXW)]],
                            rows_f.at[pl.ds(g * _IDXW, _IDXW)], sem_f).wait()
      return c
    lax.fori_loop(0, _G, drain_f, 0)
    scores_for(rows_f)
    def rowf(r, c):
      p = pool_row(r, idxf, rows_f)
      plsc.store_scatter(pool_f, [_splat(r), iota], p)
      return c
    lax.fori_loop(0, _CH, rowf, 0)

    def drain_b(g, c):
      pltpu.make_async_copy(subj_emb.at[idxb.at[pl.ds(g * _IDXW, _IDXW)]],
                            rows_b.at[pl.ds(g * _IDXW, _IDXW)], sem_b).wait()
      return c
    lax.fori_loop(0, _G, drain_b, 0)
    scores_for(rows_b)
    def rowb(r, c):
      pb = pool_row(r, idxb, rows_b)
      pf = plsc.load_gather(pool_f, [_splat(r), iota])
      d = jnp.sum(pf * pb)
      plsc.store_scatter(outc, [_splat(r)], _splat(d), mask=iota == 0)
      return c
    lax.fori_loop(0, _CH, rowb, 0)

    pltpu.make_async_copy(user_bias.at[uidxs], ub, sem_s).wait()
    pltpu.make_async_copy(item_bias.at[iidxs], ib, sem_s).wait()
    for i in range(_CH // _LANES):
      ov = outc[pl.ds(i * _LANES, _LANES)]
      ulane = lax.bitwise_and(uidx[pl.ds(i * _LANES, _LANES)], 15)
      ilane = lax.bitwise_and(iidx[pl.ds(i * _LANES, _LANES)], 15)
      ubv = plsc.load_gather(ub, [i * _LANES + iota, ulane])
      ibv = plsc.load_gather(ib, [i * _LANES + iota, ilane])
      outc[pl.ds(i * _LANES, _LANES)] = ov + ubv + ibv + gb_sp
    pltpu.sync_copy(outc, out.at[pl.ds(rbase, _CH)])
    return carry

  lax.fori_loop(0, n_chunks, chunk, 0)


def kernel(user_idx, item_idx, fav_subjects, book_subjects, subj_emb, attn_w,
           attn_b, user_bias, item_bias, global_bias):
  B = user_idx.shape[0]
  fav_r = fav_subjects.astype(jnp.int32).reshape(-1)
  book_r = book_subjects.astype(jnp.int32).reshape(-1)
  consts = jnp.concatenate([
      jnp.reshape(attn_b, (1,)).astype(jnp.float32),
      jnp.reshape(global_bias, (1,)).astype(jnp.float32),
  ])
  mesh = plsc.VectorSubcoreMesh(core_axis_name="c", subcore_axis_name="s",
                                num_cores=_NC, num_subcores=_NS)
  run = functools.partial(
      pl.kernel,
      out_type=jax.ShapeDtypeStruct((B,), jnp.float32),
      mesh=mesh,
      compiler_params=pltpu.CompilerParams(needs_layout_passes=False,
                                           use_tc_tiling_on_sc=False),
      scratch_types=[
          pltpu.VMEM((_NIDX,), jnp.int32),
          pltpu.VMEM((_NIDX,), jnp.int32),
          pltpu.VMEM((_NIDX, _D), jnp.float32),
          pltpu.VMEM((_NIDX, _D), jnp.float32),
          pltpu.VMEM((_NIDX,), jnp.float32),
          pltpu.VMEM((_CH, _D), jnp.float32),
          pltpu.VMEM((_CH,), jnp.int32),
          pltpu.VMEM((_CH,), jnp.int32),
          pltpu.VMEM((_CH,), jnp.int32),
          pltpu.VMEM((_CH,), jnp.int32),
          pltpu.VMEM((_CH, _LANES), jnp.float32),
          pltpu.VMEM((_CH, _LANES), jnp.float32),
          pltpu.VMEM((_CH,), jnp.float32),
          pltpu.VMEM((2 * _LANES,), jnp.float32),
          pltpu.VMEM((_LANES,), jnp.float32),
          pltpu.VMEM((_LANES,), jnp.int32),
          pltpu.SemaphoreType.DMA,
          pltpu.SemaphoreType.DMA,
          pltpu.SemaphoreType.DMA,
      ],
  )(_sc_body)
  z16 = jnp.zeros((_LANES,), jnp.int32)
  ub_r = user_bias.astype(jnp.float32).reshape(-1, _LANES)
  ib_r = item_bias.astype(jnp.float32).reshape(-1, _LANES)
  return run(user_idx.astype(jnp.int32), item_idx.astype(jnp.int32),
             fav_r, book_r, subj_emb.astype(jnp.float32),
             attn_w.astype(jnp.float32), consts, z16, ub_r, ib_r)
